# Initial kernel scaffold; baseline (speedup 1.0000x reference)
import jax
import jax.numpy as jnp
from jax.experimental import pallas as pl


def kernel(x, l_w, l_b, gamma, beta, W1, b1):
    raise NotImplementedError("write your pallas kernel here")


# speedup vs baseline: 29.7190x; 29.7190x over previous
import jax
import jax.numpy as jnp
from jax.experimental import pallas as pl
from jax.experimental.pallas import tpu as pltpu

L, N, C, O, TOP_K = 8, 128, 2048, 2048, 1024


def _bitonic_topk_desc(v, idx):
  R, W = v.shape
  lane = jax.lax.broadcasted_iota(jnp.int32, (R, W), 1)
  k = 2
  while k <= W:
    j = k // 2
    asc = (lane & k) != 0
    while j >= 1:
      low = (lane & j) == 0
      is_upper = ~low
      pv = jnp.where(low, pltpu.roll(v, W - j, axis=1), pltpu.roll(v, j, axis=1))
      pi = jnp.where(low, pltpu.roll(idx, W - j, axis=1), pltpu.roll(idx, j, axis=1))
      self_first = (v > pv) | ((v == pv) & (idx < pi))
      take_self = self_first ^ is_upper ^ asc
      v = jnp.where(take_self, v, pv)
      idx = jnp.where(take_self, idx, pi)
      j //= 2
    k *= 2
  return v, idx


def _gate_kernel(x_norm_ref, l_w_ref, l_b_ref, W1_ref, b1_ref,
                 idx_ref, mask_ref):
  x_norm = x_norm_ref[...]
  l_w = l_w_ref[...]
  l_b = l_b_ref[...]
  W1 = W1_ref[...]
  b1 = b1_ref[...]

  dn = (((1,), (1,)), ((), ()))
  gpi = jax.lax.dot_general(x_norm, l_w, dn,
                            preferred_element_type=jnp.float32) + l_b
  logits = jax.lax.dot_general(gpi, W1, dn,
                               preferred_element_type=jnp.float32) + b1
  logits = jnp.maximum(logits, 0.0)

  iota = jax.lax.broadcasted_iota(jnp.int32, (N, O), 1)
  sv, si = _bitonic_topk_desc(logits, iota)
  idx_ref[...] = si[:, :TOP_K]

  v_thr = sv[:, TOP_K - 1:TOP_K]
  i_thr = si[:, TOP_K - 1:TOP_K]
  mask = (logits > v_thr) | ((logits == v_thr) & (iota <= i_thr))
  mask_ref[...] = mask.astype(jnp.float32)


def _dense_kernel(x_ref, l_w_ref, l_b_ref, mask_ref, out_ref):
  x2 = x_ref[...].reshape(L * N, C)
  dn = (((1,), (1,)), ((), ()))
  dense = jax.lax.dot_general(x2, l_w_ref[...], dn,
                              preferred_element_type=jnp.float32) + l_b_ref[...]
  bo = dense.shape[1]
  out_ref[...] = dense.reshape(L, N, bo) * mask_ref[...][None]


_BO = 512


@jax.jit
def kernel(x, l_w, l_b, gamma, beta, W1, b1):
  xp = jnp.transpose(x, (1, 0, 2))
  x_avg = jnp.mean(xp, axis=1)
  mu = jnp.mean(x_avg, axis=-1, keepdims=True)
  var = jnp.mean((x_avg - mu) ** 2, axis=-1, keepdims=True)
  x_norm = (x_avg - mu) / jnp.sqrt(var + 1e-5) * gamma + beta

  l_b2 = l_b.reshape(1, O)
  idx, mask = pl.pallas_call(
      _gate_kernel,
      out_shape=(
          jax.ShapeDtypeStruct((N, TOP_K), jnp.int32),
          jax.ShapeDtypeStruct((N, O), jnp.float32),
      ),
  )(x_norm, l_w, l_b2, W1, b1.reshape(1, O))

  out = pl.pallas_call(
      _dense_kernel,
      grid=(O // _BO,),
      in_specs=[
          pl.BlockSpec((L, N, C), lambda o: (0, 0, 0)),
          pl.BlockSpec((_BO, C), lambda o: (o, 0)),
          pl.BlockSpec((1, _BO), lambda o: (0, o)),
          pl.BlockSpec((N, _BO), lambda o: (0, o)),
      ],
      out_specs=pl.BlockSpec((L, N, _BO), lambda o: (0, 0, o)),
      out_shape=jax.ShapeDtypeStruct((L, N, O), jnp.float32),
  )(x, l_w, l_b2, mask)
  return out, idx
